# baseline (speedup 1.0000x reference)
import jax
import jax.numpy as jnp
from jax.experimental import pallas as pl
from jax.experimental.pallas import tpu as pltpu

_BLOCK_ROWS = 12544


def _copy_body(in_ref, out_ref):
    out_ref[...] = in_ref[...]


def kernel(input):
    b, c, h, w = input.shape
    flat = jnp.transpose(input, (0, 2, 3, 1)).reshape(b * h * w, c)
    rows = flat.shape[0]
    out = pl.pallas_call(
        _copy_body,
        out_shape=jax.ShapeDtypeStruct(flat.shape, flat.dtype),
        grid=(rows // _BLOCK_ROWS,),
        in_specs=[pl.BlockSpec((_BLOCK_ROWS, c), lambda i: (i, 0))],
        out_specs=pl.BlockSpec((_BLOCK_ROWS, c), lambda i: (i, 0)),
    )(flat)
    return jnp.transpose(out.reshape(b, h, w, c), (0, 3, 1, 2))
